# baseline (speedup 1.0000x reference)
import functools

import jax
import jax.numpy as jnp
from jax import lax
from jax.experimental import pallas as pl
from jax.experimental.pallas import tpu as pltpu
from jax.experimental.pallas import tpu_sc as plsc

N_NODES = 10000
N_EDGES = 160000
D = 42
DP = 48
DB = 64
KP = 43 * DP
STEPS = 3

SC_CORES = 2
SC_SUBCORES = 16
SC_WORKERS = SC_CORES * SC_SUBCORES

SCAT_ROW = 125
SCAT_NROWS = N_EDGES // SCAT_ROW
SCAT_WROWS = SCAT_NROWS // SC_WORKERS
SCAT_GRP = 10
SCAT_NGRP = SCAT_WROWS // SCAT_GRP

GATHER_CHUNK = 128
GATHER_MULT = 5
GATHER_WIN = GATHER_CHUNK * GATHER_MULT

BE = 1280
BN = 1000

_F32 = jnp.float32


def _proj_body(x_ref, w_ref, b_ref, o_ref, ob_ref):
    val = jnp.maximum(
        jnp.dot(x_ref[...], w_ref[...], preferred_element_type=_F32)
        + b_ref[...], 0.0)
    o_ref[...] = val
    ob_ref[...] = jnp.pad(val, ((0, 0), (0, DB - DP))).astype(jnp.bfloat16)


def _proj(x, w, b, blk):
    rows = x.shape[0]
    return pl.pallas_call(
        _proj_body,
        grid=(rows // blk,),
        in_specs=[
            pl.BlockSpec((blk, x.shape[1]), lambda i: (i, 0)),
            pl.BlockSpec(w.shape, lambda i: (0, 0)),
            pl.BlockSpec(b.shape, lambda i: (0, 0)),
        ],
        out_specs=[pl.BlockSpec((blk, DP), lambda i: (i, 0)),
                   pl.BlockSpec((blk, DB), lambda i: (i, 0))],
        out_shape=[jax.ShapeDtypeStruct((rows, DP), _F32),
                   jax.ShapeDtypeStruct((rows, DB), jnp.bfloat16)],
    )(x, w, b)


def _proj_t_body(x_ref, w_ref, b_ref, o_ref):
    o_ref[...] = jnp.maximum(
        jnp.dot(x_ref[...], w_ref[...], preferred_element_type=_F32)
        + b_ref[...], 0.0).T


def _proj_t(x, w, b, blk):
    rows = x.shape[0]
    return pl.pallas_call(
        _proj_t_body,
        grid=(rows // blk,),
        in_specs=[
            pl.BlockSpec((blk, x.shape[1]), lambda i: (i, 0)),
            pl.BlockSpec(w.shape, lambda i: (0, 0)),
            pl.BlockSpec(b.shape, lambda i: (0, 0)),
        ],
        out_specs=pl.BlockSpec((DP, blk), lambda i: (0, i)),
        out_shape=jax.ShapeDtypeStruct((DP, rows), _F32),
    )(x, w, b)


def _msg_body(ht_ref, xj_ref, w2t_ref, o_ref):
    ut = lax.dot_general(w2t_ref[...], xj_ref[...],
                         (((1,), (1,)), ((), ())),
                         preferred_element_type=_F32)
    u3 = ut.reshape(43, DP, BE)
    ht = ht_ref[...]
    acc = u3[0] * ht[0][None, :]
    for k in range(1, 43):
        acc = acc + u3[k] * ht[k][None, :]
    o_ref[...] = acc.T


def _msgs(ht, xj, w2t):
    return pl.pallas_call(
        _msg_body,
        grid=(N_EDGES // BE,),
        in_specs=[
            pl.BlockSpec((DP, BE), lambda i: (0, i)),
            pl.BlockSpec((BE, DB), lambda i: (i, 0)),
            pl.BlockSpec((KP, DB), lambda i: (0, 0)),
        ],
        out_specs=pl.BlockSpec((BE, DP), lambda i: (i, 0)),
        out_shape=jax.ShapeDtypeStruct((N_EDGES, DP), _F32),
    )(ht, xj, w2t)


def _upd_core(ap_ref, prev_ref, root_ref, cb_ref, w1_ref, w2_ref, mb_ref):
    aggr = ap_ref[0] + ap_ref[1]
    prev = prev_ref[...]
    conv = aggr + jnp.dot(prev, root_ref[...], preferred_element_type=_F32) + cb_ref[...]
    m = jnp.maximum(conv, 0.0)
    return (jnp.dot(m, w1_ref[...], preferred_element_type=_F32)
            + jnp.dot(prev, w2_ref[...], preferred_element_type=_F32)
            + mb_ref[...])


def _upd_body(ap_ref, prev_ref, root_ref, cb_ref, w1_ref, w2_ref, mb_ref,
              o_ref, ob_ref):
    val = _upd_core(ap_ref, prev_ref, root_ref, cb_ref, w1_ref, w2_ref, mb_ref)
    o_ref[...] = val
    ob_ref[...] = jnp.pad(val, ((0, 0), (0, DB - DP))).astype(jnp.bfloat16)


def _upd_final_body(ap_ref, prev_ref, root_ref, cb_ref, w1_ref, w2_ref, mb_ref,
                    init_ref, o_ref):
    full = _upd_core(ap_ref, prev_ref, root_ref, cb_ref, w1_ref, w2_ref, mb_ref)
    o_ref[...] = full[:, :D] + init_ref[...]


def _upd_weight_specs():
    return [
        pl.BlockSpec((2, BN, DP), lambda i: (0, i, 0)),
        pl.BlockSpec((BN, DP), lambda i: (i, 0)),
        pl.BlockSpec((DP, DP), lambda i: (0, 0)),
        pl.BlockSpec((1, DP), lambda i: (0, 0)),
        pl.BlockSpec((DP, DP), lambda i: (0, 0)),
        pl.BlockSpec((DP, DP), lambda i: (0, 0)),
        pl.BlockSpec((1, DP), lambda i: (0, 0)),
    ]


def _update(parts, prev, rootp, cbp, mw1, mw2, mbp):
    return pl.pallas_call(
        _upd_body,
        grid=(N_NODES // BN,),
        in_specs=_upd_weight_specs(),
        out_specs=[pl.BlockSpec((BN, DP), lambda i: (i, 0)),
                   pl.BlockSpec((BN, DB), lambda i: (i, 0))],
        out_shape=[jax.ShapeDtypeStruct((N_NODES, DP), _F32),
                   jax.ShapeDtypeStruct((N_NODES, DB), jnp.bfloat16)],
    )(parts, prev, rootp, cbp, mw1, mw2, mbp)


def _update_final(parts, prev, rootp, cbp, mw1, mw2, mbp, init):
    return pl.pallas_call(
        _upd_final_body,
        grid=(N_NODES // BN,),
        in_specs=_upd_weight_specs() + [pl.BlockSpec((BN, D), lambda i: (i, 0))],
        out_specs=pl.BlockSpec((BN, D), lambda i: (i, 0)),
        out_shape=jax.ShapeDtypeStruct((N_NODES, D), _F32),
    )(parts, prev, rootp, cbp, mw1, mw2, mbp, init)


def _sc_mesh():
    return plsc.VectorSubcoreMesh(core_axis_name="core", subcore_axis_name="subcore")


_SC_PARAMS = pltpu.CompilerParams(use_tc_tiling_on_sc=False)


def _sc_gather(table, idx2d):
    @functools.partial(
        pl.kernel,
        out_type=jax.ShapeDtypeStruct((N_EDGES, DB), jnp.bfloat16),
        mesh=_sc_mesh(),
        compiler_params=_SC_PARAMS,
    )
    def k(tab_hbm, i_hbm, o_hbm):
        def body(i_vmem, o_vmem):
            for m in range(GATHER_MULT):
                pltpu.sync_copy(tab_hbm.at[i_vmem.at[m]],
                                o_vmem.at[pl.ds(m * GATHER_CHUNK, GATHER_CHUNK)])

        pltpu.emit_pipeline(
            body,
            grid=(N_EDGES // GATHER_WIN,),
            in_specs=[pl.BlockSpec((GATHER_MULT, GATHER_CHUNK), lambda i: (i, 0))],
            out_specs=[pl.BlockSpec((GATHER_WIN, DB), lambda i: (i, 0))],
            core_axis_name=("core", "subcore"),
            dimension_semantics=(pltpu.PARALLEL,),
        )(i_hbm, o_hbm)

    return k(table, idx2d)


def _sc_scatter(msgs, dst3, zeros_nd):
    @functools.partial(
        pl.kernel,
        out_type=jax.ShapeDtypeStruct((SC_CORES, N_NODES, DP), _F32),
        mesh=_sc_mesh(),
        scratch_types=[
            pltpu.VMEM((SCAT_GRP, SCAT_ROW), jnp.int32),
            pltpu.VMEM((SCAT_GRP * SCAT_ROW, DP), _F32),
            pltpu.VMEM_SHARED((N_NODES, DP), _F32),
        ],
        compiler_params=_SC_PARAMS,
    )
    def k(msgs_hbm, dst_hbm, z_hbm, o_hbm, idx_v, rows_v, aggr_sh):
        c = lax.axis_index("core")
        s = lax.axis_index("subcore")
        stripe = N_NODES // SC_SUBCORES

        pltpu.sync_copy(z_hbm.at[pl.ds(s * stripe, stripe)],
                        aggr_sh.at[pl.ds(s * stripe, stripe)])
        plsc.subcore_barrier()

        w = c * SC_SUBCORES + s
        row0 = w * SCAT_WROWS

        @pl.loop(0, SCAT_NGRP)
        def _(g):
            base = row0 + g * SCAT_GRP
            pltpu.sync_copy(dst_hbm.at[pl.ds(base, SCAT_GRP)], idx_v)
            pltpu.sync_copy(msgs_hbm.at[pl.ds(base * SCAT_ROW, SCAT_GRP * SCAT_ROW)],
                            rows_v)
            for m in range(SCAT_GRP):
                pltpu.sync_copy(rows_v.at[pl.ds(m * SCAT_ROW, SCAT_ROW)],
                                aggr_sh.at[idx_v.at[m]], add=True)

        plsc.subcore_barrier()
        pltpu.sync_copy(aggr_sh.at[pl.ds(s * stripe, stripe)],
                        o_hbm.at[c, pl.ds(s * stripe, stripe)])

    return k(msgs, dst3, zeros_nd)


def kernel(x, edge_index, edge_attr, lin0_W, lin0_b, en1_W, en1_b, en2_W, en2_b,
           root, conv_b, msg_W, msg_b):
    f32 = _F32
    pad_w = lambda w: jnp.pad(w.astype(f32), ((0, DP - w.shape[0]), (0, DP - w.shape[1])))
    pad_b = lambda b: jnp.pad(b.astype(f32), (0, DP - b.shape[0]))[None, :]

    w0 = jnp.pad(lin0_W, ((0, 0), (0, DP - D)))
    b0 = pad_b(lin0_b)
    w1 = jnp.pad(en1_W, ((0, 0), (0, DP - D)))
    b1 = pad_b(en1_b).at[0, D].set(1.0)

    ct = en2_W.reshape(D, D, D).transpose(0, 2, 1)
    ck = jnp.pad(ct, ((0, 0), (0, DP - D), (0, DB - D)))
    cb = jnp.pad(en2_b.reshape(D, D).T, ((0, DP - D), (0, DB - D)))[None]
    w2t = jnp.concatenate([ck, cb], axis=0).reshape(KP, DB).astype(jnp.bfloat16)

    rootp = pad_w(root)
    cbp = pad_b(conv_b)
    mw1 = pad_w(msg_W[:D])
    mw2 = pad_w(msg_W[D:])
    mbp = pad_b(msg_b)

    src2d = edge_index[0].reshape(N_EDGES // GATHER_CHUNK, GATHER_CHUNK)
    dst3 = edge_index[1].reshape(SCAT_NROWS, SCAT_ROW)
    zeros_nd = jnp.zeros((N_NODES, DP), f32)

    out, out_b = _proj(x, w0, b0, BN)
    ht = _proj_t(edge_attr, w1, b1, BE)

    for step in range(STEPS):
        xj = _sc_gather(out_b, src2d)
        msgs = _msgs(ht, xj, w2t)
        parts = _sc_scatter(msgs, dst3, zeros_nd)
        if step < STEPS - 1:
            out, out_b = _update(parts, out, rootp, cbp, mw1, mw2, mbp)
        else:
            out = _update_final(parts, out, rootp, cbp, mw1, mw2, mbp, x)
    return out


# speedup vs baseline: 1.0817x; 1.0817x over previous
import functools

import jax
import jax.numpy as jnp
from jax import lax
from jax.experimental import pallas as pl
from jax.experimental.pallas import tpu as pltpu
from jax.experimental.pallas import tpu_sc as plsc

N_NODES = 10000
N_EDGES = 160000
D = 42
DP = 48
DB = 64
KP = 43 * DP
STEPS = 3

SC_CORES = 2
SC_SUBCORES = 16
SC_WORKERS = SC_CORES * SC_SUBCORES

SCAT_ROW = 125
SCAT_NROWS = N_EDGES // SCAT_ROW
SCAT_WROWS = SCAT_NROWS // SC_WORKERS
SCAT_GRP = 10
SCAT_NGRP = SCAT_WROWS // SCAT_GRP

GATHER_CHUNK = 128
GATHER_MULT = 5
GATHER_WIN = GATHER_CHUNK * GATHER_MULT

BE = 1280
BN = 1000

_F32 = jnp.float32


def _proj_body(x_ref, w_ref, b_ref, o_ref):
    o_ref[...] = jnp.maximum(
        jnp.dot(x_ref[...], w_ref[...], preferred_element_type=_F32)
        + b_ref[...], 0.0)


def _proj(x, w, b, blk):
    rows = x.shape[0]
    return pl.pallas_call(
        _proj_body,
        grid=(rows // blk,),
        in_specs=[
            pl.BlockSpec((blk, x.shape[1]), lambda i: (i, 0)),
            pl.BlockSpec(w.shape, lambda i: (0, 0)),
            pl.BlockSpec(b.shape, lambda i: (0, 0)),
        ],
        out_specs=pl.BlockSpec((blk, DP), lambda i: (i, 0)),
        out_shape=jax.ShapeDtypeStruct((rows, DP), _F32),
    )(x, w, b)


def _proj_t_body(x_ref, w_ref, b_ref, o_ref):
    o_ref[...] = jnp.maximum(
        jnp.dot(x_ref[...], w_ref[...], preferred_element_type=_F32)
        + b_ref[...], 0.0).T


def _proj_t(x, w, b, blk):
    rows = x.shape[0]
    return pl.pallas_call(
        _proj_t_body,
        grid=(rows // blk,),
        in_specs=[
            pl.BlockSpec((blk, x.shape[1]), lambda i: (i, 0)),
            pl.BlockSpec(w.shape, lambda i: (0, 0)),
            pl.BlockSpec(b.shape, lambda i: (0, 0)),
        ],
        out_specs=pl.BlockSpec((DP, blk), lambda i: (0, i)),
        out_shape=jax.ShapeDtypeStruct((DP, rows), _F32),
    )(x, w, b)


def _msg_body(ht_ref, xj_ref, w2t_ref, o_ref):
    ut = lax.dot_general(w2t_ref[...], xj_ref[...],
                         (((1,), (1,)), ((), ())),
                         preferred_element_type=_F32)
    u3 = ut.reshape(43, DP, BE)
    ht = ht_ref[...]
    acc = u3[0] * ht[0][None, :]
    for k in range(1, 43):
        acc = acc + u3[k] * ht[k][None, :]
    o_ref[...] = acc.T


def _msgs(ht, xj, w2t):
    return pl.pallas_call(
        _msg_body,
        grid=(N_EDGES // BE,),
        in_specs=[
            pl.BlockSpec((DP, BE), lambda i: (0, i)),
            pl.BlockSpec((BE, DP), lambda i: (i, 0)),
            pl.BlockSpec((KP, DP), lambda i: (0, 0)),
        ],
        out_specs=pl.BlockSpec((BE, DP), lambda i: (i, 0)),
        out_shape=jax.ShapeDtypeStruct((N_EDGES, DP), _F32),
    )(ht, xj, w2t)


def _upd_core(ap_ref, prev_ref, root_ref, cb_ref, w1_ref, w2_ref, mb_ref):
    aggr = ap_ref[0] + ap_ref[1]
    prev = prev_ref[...]
    conv = aggr + jnp.dot(prev, root_ref[...], preferred_element_type=_F32) + cb_ref[...]
    m = jnp.maximum(conv, 0.0)
    return (jnp.dot(m, w1_ref[...], preferred_element_type=_F32)
            + jnp.dot(prev, w2_ref[...], preferred_element_type=_F32)
            + mb_ref[...])


def _upd_body(ap_ref, prev_ref, root_ref, cb_ref, w1_ref, w2_ref, mb_ref, o_ref):
    o_ref[...] = _upd_core(ap_ref, prev_ref, root_ref, cb_ref, w1_ref, w2_ref, mb_ref)


def _upd_final_body(ap_ref, prev_ref, root_ref, cb_ref, w1_ref, w2_ref, mb_ref,
                    init_ref, o_ref):
    full = _upd_core(ap_ref, prev_ref, root_ref, cb_ref, w1_ref, w2_ref, mb_ref)
    o_ref[...] = full[:, :D] + init_ref[...]


def _upd_weight_specs():
    return [
        pl.BlockSpec((2, BN, DP), lambda i: (0, i, 0)),
        pl.BlockSpec((BN, DP), lambda i: (i, 0)),
        pl.BlockSpec((DP, DP), lambda i: (0, 0)),
        pl.BlockSpec((1, DP), lambda i: (0, 0)),
        pl.BlockSpec((DP, DP), lambda i: (0, 0)),
        pl.BlockSpec((DP, DP), lambda i: (0, 0)),
        pl.BlockSpec((1, DP), lambda i: (0, 0)),
    ]


def _update(parts, prev, rootp, cbp, mw1, mw2, mbp):
    return pl.pallas_call(
        _upd_body,
        grid=(N_NODES // BN,),
        in_specs=_upd_weight_specs(),
        out_specs=pl.BlockSpec((BN, DP), lambda i: (i, 0)),
        out_shape=jax.ShapeDtypeStruct((N_NODES, DP), _F32),
    )(parts, prev, rootp, cbp, mw1, mw2, mbp)


def _update_final(parts, prev, rootp, cbp, mw1, mw2, mbp, init):
    return pl.pallas_call(
        _upd_final_body,
        grid=(N_NODES // BN,),
        in_specs=_upd_weight_specs() + [pl.BlockSpec((BN, D), lambda i: (i, 0))],
        out_specs=pl.BlockSpec((BN, D), lambda i: (i, 0)),
        out_shape=jax.ShapeDtypeStruct((N_NODES, D), _F32),
    )(parts, prev, rootp, cbp, mw1, mw2, mbp, init)


def _sc_mesh():
    return plsc.VectorSubcoreMesh(core_axis_name="core", subcore_axis_name="subcore")


_SC_PARAMS = pltpu.CompilerParams(use_tc_tiling_on_sc=False)


def _sc_gather(table, idx2d):
    @functools.partial(
        pl.kernel,
        out_type=jax.ShapeDtypeStruct((N_EDGES, DP), _F32),
        mesh=_sc_mesh(),
        compiler_params=_SC_PARAMS,
    )
    def k(tab_hbm, i_hbm, o_hbm):
        def body(i_vmem, o_vmem):
            for m in range(GATHER_MULT):
                pltpu.sync_copy(tab_hbm.at[i_vmem.at[m]],
                                o_vmem.at[pl.ds(m * GATHER_CHUNK, GATHER_CHUNK)])

        pltpu.emit_pipeline(
            body,
            grid=(N_EDGES // GATHER_WIN,),
            in_specs=[pl.BlockSpec((GATHER_MULT, GATHER_CHUNK), lambda i: (i, 0))],
            out_specs=[pl.BlockSpec((GATHER_WIN, DP), lambda i: (i, 0))],
            core_axis_name=("core", "subcore"),
            dimension_semantics=(pltpu.PARALLEL,),
        )(i_hbm, o_hbm)

    return k(table, idx2d)


def _sc_scatter(msgs, dst3, zeros_nd):
    @functools.partial(
        pl.kernel,
        out_type=jax.ShapeDtypeStruct((SC_CORES, N_NODES, DP), _F32),
        mesh=_sc_mesh(),
        scratch_types=[
            pltpu.VMEM((SCAT_GRP, SCAT_ROW), jnp.int32),
            pltpu.VMEM((SCAT_GRP * SCAT_ROW, DP), _F32),
            pltpu.VMEM_SHARED((N_NODES, DP), _F32),
        ],
        compiler_params=_SC_PARAMS,
    )
    def k(msgs_hbm, dst_hbm, z_hbm, o_hbm, idx_v, rows_v, aggr_sh):
        c = lax.axis_index("core")
        s = lax.axis_index("subcore")
        stripe = N_NODES // SC_SUBCORES

        pltpu.sync_copy(z_hbm.at[pl.ds(s * stripe, stripe)],
                        aggr_sh.at[pl.ds(s * stripe, stripe)])
        plsc.subcore_barrier()

        w = c * SC_SUBCORES + s
        row0 = w * SCAT_WROWS

        @pl.loop(0, SCAT_NGRP)
        def _(g):
            base = row0 + g * SCAT_GRP
            pltpu.sync_copy(dst_hbm.at[pl.ds(base, SCAT_GRP)], idx_v)
            pltpu.sync_copy(msgs_hbm.at[pl.ds(base * SCAT_ROW, SCAT_GRP * SCAT_ROW)],
                            rows_v)
            for m in range(SCAT_GRP):
                pltpu.sync_copy(rows_v.at[pl.ds(m * SCAT_ROW, SCAT_ROW)],
                                aggr_sh.at[idx_v.at[m]], add=True)

        plsc.subcore_barrier()
        pltpu.sync_copy(aggr_sh.at[pl.ds(s * stripe, stripe)],
                        o_hbm.at[c, pl.ds(s * stripe, stripe)])

    return k(msgs, dst3, zeros_nd)


def kernel(x, edge_index, edge_attr, lin0_W, lin0_b, en1_W, en1_b, en2_W, en2_b,
           root, conv_b, msg_W, msg_b):
    f32 = _F32
    pad_w = lambda w: jnp.pad(w.astype(f32), ((0, DP - w.shape[0]), (0, DP - w.shape[1])))
    pad_b = lambda b: jnp.pad(b.astype(f32), (0, DP - b.shape[0]))[None, :]

    w0 = jnp.pad(lin0_W, ((0, 0), (0, DP - D)))
    b0 = pad_b(lin0_b)
    w1 = jnp.pad(en1_W, ((0, 0), (0, DP - D)))
    b1 = pad_b(en1_b).at[0, D].set(1.0)

    ct = en2_W.reshape(D, D, D).transpose(0, 2, 1)
    ck = jnp.pad(ct, ((0, 0), (0, DP - D), (0, DP - D)))
    cb = jnp.pad(en2_b.reshape(D, D).T, ((0, DP - D), (0, DP - D)))[None]
    w2t = jnp.concatenate([ck, cb], axis=0).reshape(KP, DP)

    rootp = pad_w(root)
    cbp = pad_b(conv_b)
    mw1 = pad_w(msg_W[:D])
    mw2 = pad_w(msg_W[D:])
    mbp = pad_b(msg_b)

    src2d = edge_index[0].reshape(N_EDGES // GATHER_CHUNK, GATHER_CHUNK)
    dst3 = edge_index[1].reshape(SCAT_NROWS, SCAT_ROW)
    zeros_nd = jnp.zeros((N_NODES, DP), f32)

    out = _proj(x, w0, b0, BN)
    ht = _proj_t(edge_attr, w1, b1, BE)

    for step in range(STEPS):
        xj = _sc_gather(out, src2d)
        msgs = _msgs(ht, xj, w2t)
        parts = _sc_scatter(msgs, dst3, zeros_nd)
        if step < STEPS - 1:
            out = _update(parts, out, rootp, cbp, mw1, mw2, mbp)
        else:
            out = _update_final(parts, out, rootp, cbp, mw1, mw2, mbp, x)
    return out


# speedup vs baseline: 1.1769x; 1.0879x over previous
import functools

import jax
import jax.numpy as jnp
from jax import lax
from jax.experimental import pallas as pl
from jax.experimental.pallas import tpu as pltpu
from jax.experimental.pallas import tpu_sc as plsc

N_NODES = 10000
N_EDGES = 160000
D = 42
DP = 48
DB = 64
KP = 43 * DP
STEPS = 3

SC_CORES = 2
SC_SUBCORES = 16
SC_WORKERS = SC_CORES * SC_SUBCORES

SCAT_ROW = 125
SCAT_NROWS = N_EDGES // SCAT_ROW
SCAT_WROWS = SCAT_NROWS // SC_WORKERS
SCAT_GRP = 10
SCAT_NGRP = SCAT_WROWS // SCAT_GRP

GATHER_CHUNK = 128
GATHER_MULT = 5
GATHER_WIN = GATHER_CHUNK * GATHER_MULT

BE = 3200
BN = 1000

_F32 = jnp.float32


def _proj_body(x_ref, w_ref, b_ref, o_ref):
    o_ref[...] = jnp.maximum(
        jnp.dot(x_ref[...], w_ref[...], preferred_element_type=_F32)
        + b_ref[...], 0.0)


def _proj(x, w, b, blk):
    rows = x.shape[0]
    return pl.pallas_call(
        _proj_body,
        grid=(rows // blk,),
        in_specs=[
            pl.BlockSpec((blk, x.shape[1]), lambda i: (i, 0)),
            pl.BlockSpec(w.shape, lambda i: (0, 0)),
            pl.BlockSpec(b.shape, lambda i: (0, 0)),
        ],
        out_specs=pl.BlockSpec((blk, DP), lambda i: (i, 0)),
        out_shape=jax.ShapeDtypeStruct((rows, DP), _F32),
    )(x, w, b)


def _proj_t_body(x_ref, w_ref, b_ref, o_ref):
    o_ref[...] = jnp.maximum(
        jnp.dot(x_ref[...], w_ref[...], preferred_element_type=_F32)
        + b_ref[...], 0.0).T


def _proj_t(x, w, b, blk):
    rows = x.shape[0]
    return pl.pallas_call(
        _proj_t_body,
        grid=(rows // blk,),
        in_specs=[
            pl.BlockSpec((blk, x.shape[1]), lambda i: (i, 0)),
            pl.BlockSpec(w.shape, lambda i: (0, 0)),
            pl.BlockSpec(b.shape, lambda i: (0, 0)),
        ],
        out_specs=pl.BlockSpec((DP, blk), lambda i: (0, i)),
        out_shape=jax.ShapeDtypeStruct((DP, rows), _F32),
    )(x, w, b)


def _msg_body(ht_ref, xj_ref, w2t_ref, o_ref):
    ut = lax.dot_general(w2t_ref[...], xj_ref[...],
                         (((1,), (1,)), ((), ())),
                         preferred_element_type=_F32)
    u3 = ut.reshape(43, DP, BE)
    ht = ht_ref[...]
    acc = u3[0] * ht[0][None, :]
    for k in range(1, 43):
        acc = acc + u3[k] * ht[k][None, :]
    o_ref[...] = acc.T


def _msgs(ht, xj, w2t):
    return pl.pallas_call(
        _msg_body,
        grid=(N_EDGES // BE,),
        in_specs=[
            pl.BlockSpec((DP, BE), lambda i: (0, i)),
            pl.BlockSpec((BE, DP), lambda i: (i, 0)),
            pl.BlockSpec((KP, DP), lambda i: (0, 0)),
        ],
        out_specs=pl.BlockSpec((BE, DP), lambda i: (i, 0)),
        out_shape=jax.ShapeDtypeStruct((N_EDGES, DP), _F32),
    )(ht, xj, w2t)


def _upd_core(ap_ref, prev_ref, root_ref, cb_ref, w1_ref, w2_ref, mb_ref):
    aggr = ap_ref[0] + ap_ref[1]
    prev = prev_ref[...]
    conv = aggr + jnp.dot(prev, root_ref[...], preferred_element_type=_F32) + cb_ref[...]
    m = jnp.maximum(conv, 0.0)
    return (jnp.dot(m, w1_ref[...], preferred_element_type=_F32)
            + jnp.dot(prev, w2_ref[...], preferred_element_type=_F32)
            + mb_ref[...])


def _upd_body(ap_ref, prev_ref, root_ref, cb_ref, w1_ref, w2_ref, mb_ref, o_ref):
    o_ref[...] = _upd_core(ap_ref, prev_ref, root_ref, cb_ref, w1_ref, w2_ref, mb_ref)


def _upd_final_body(ap_ref, prev_ref, root_ref, cb_ref, w1_ref, w2_ref, mb_ref,
                    init_ref, o_ref):
    full = _upd_core(ap_ref, prev_ref, root_ref, cb_ref, w1_ref, w2_ref, mb_ref)
    o_ref[...] = full[:, :D] + init_ref[...]


def _upd_weight_specs():
    return [
        pl.BlockSpec((2, BN, DP), lambda i: (0, i, 0)),
        pl.BlockSpec((BN, DP), lambda i: (i, 0)),
        pl.BlockSpec((DP, DP), lambda i: (0, 0)),
        pl.BlockSpec((1, DP), lambda i: (0, 0)),
        pl.BlockSpec((DP, DP), lambda i: (0, 0)),
        pl.BlockSpec((DP, DP), lambda i: (0, 0)),
        pl.BlockSpec((1, DP), lambda i: (0, 0)),
    ]


def _update(parts, prev, rootp, cbp, mw1, mw2, mbp):
    return pl.pallas_call(
        _upd_body,
        grid=(N_NODES // BN,),
        in_specs=_upd_weight_specs(),
        out_specs=pl.BlockSpec((BN, DP), lambda i: (i, 0)),
        out_shape=jax.ShapeDtypeStruct((N_NODES, DP), _F32),
    )(parts, prev, rootp, cbp, mw1, mw2, mbp)


def _update_final(parts, prev, rootp, cbp, mw1, mw2, mbp, init):
    return pl.pallas_call(
        _upd_final_body,
        grid=(N_NODES // BN,),
        in_specs=_upd_weight_specs() + [pl.BlockSpec((BN, D), lambda i: (i, 0))],
        out_specs=pl.BlockSpec((BN, D), lambda i: (i, 0)),
        out_shape=jax.ShapeDtypeStruct((N_NODES, D), _F32),
    )(parts, prev, rootp, cbp, mw1, mw2, mbp, init)


def _sc_mesh():
    return plsc.VectorSubcoreMesh(core_axis_name="core", subcore_axis_name="subcore")


_SC_PARAMS = pltpu.CompilerParams(use_tc_tiling_on_sc=False)


def _sc_gather(table, idx2d):
    @functools.partial(
        pl.kernel,
        out_type=jax.ShapeDtypeStruct((N_EDGES, DP), _F32),
        mesh=_sc_mesh(),
        compiler_params=_SC_PARAMS,
    )
    def k(tab_hbm, i_hbm, o_hbm):
        def body(i_vmem, o_vmem):
            for m in range(GATHER_MULT):
                pltpu.sync_copy(tab_hbm.at[i_vmem.at[m]],
                                o_vmem.at[pl.ds(m * GATHER_CHUNK, GATHER_CHUNK)])

        pltpu.emit_pipeline(
            body,
            grid=(N_EDGES // GATHER_WIN,),
            in_specs=[pl.BlockSpec((GATHER_MULT, GATHER_CHUNK), lambda i: (i, 0))],
            out_specs=[pl.BlockSpec((GATHER_WIN, DP), lambda i: (i, 0))],
            core_axis_name=("core", "subcore"),
            dimension_semantics=(pltpu.PARALLEL,),
        )(i_hbm, o_hbm)

    return k(table, idx2d)


def _sc_scatter(msgs, dst3, zeros_nd):
    @functools.partial(
        pl.kernel,
        out_type=jax.ShapeDtypeStruct((SC_CORES, N_NODES, DP), _F32),
        mesh=_sc_mesh(),
        scratch_types=[
            pltpu.VMEM((SCAT_GRP, SCAT_ROW), jnp.int32),
            pltpu.VMEM((SCAT_GRP * SCAT_ROW, DP), _F32),
            pltpu.VMEM_SHARED((N_NODES, DP), _F32),
        ],
        compiler_params=_SC_PARAMS,
    )
    def k(msgs_hbm, dst_hbm, z_hbm, o_hbm, idx_v, rows_v, aggr_sh):
        c = lax.axis_index("core")
        s = lax.axis_index("subcore")
        stripe = N_NODES // SC_SUBCORES

        pltpu.sync_copy(z_hbm.at[pl.ds(s * stripe, stripe)],
                        aggr_sh.at[pl.ds(s * stripe, stripe)])
        plsc.subcore_barrier()

        w = c * SC_SUBCORES + s
        row0 = w * SCAT_WROWS

        @pl.loop(0, SCAT_NGRP)
        def _(g):
            base = row0 + g * SCAT_GRP
            pltpu.sync_copy(dst_hbm.at[pl.ds(base, SCAT_GRP)], idx_v)
            pltpu.sync_copy(msgs_hbm.at[pl.ds(base * SCAT_ROW, SCAT_GRP * SCAT_ROW)],
                            rows_v)
            for m in range(SCAT_GRP):
                pltpu.sync_copy(rows_v.at[pl.ds(m * SCAT_ROW, SCAT_ROW)],
                                aggr_sh.at[idx_v.at[m]], add=True)

        plsc.subcore_barrier()
        pltpu.sync_copy(aggr_sh.at[pl.ds(s * stripe, stripe)],
                        o_hbm.at[c, pl.ds(s * stripe, stripe)])

    return k(msgs, dst3, zeros_nd)


def kernel(x, edge_index, edge_attr, lin0_W, lin0_b, en1_W, en1_b, en2_W, en2_b,
           root, conv_b, msg_W, msg_b):
    f32 = _F32
    pad_w = lambda w: jnp.pad(w.astype(f32), ((0, DP - w.shape[0]), (0, DP - w.shape[1])))
    pad_b = lambda b: jnp.pad(b.astype(f32), (0, DP - b.shape[0]))[None, :]

    w0 = jnp.pad(lin0_W, ((0, 0), (0, DP - D)))
    b0 = pad_b(lin0_b)
    w1 = jnp.pad(en1_W, ((0, 0), (0, DP - D)))
    b1 = pad_b(en1_b).at[0, D].set(1.0)

    ct = en2_W.reshape(D, D, D).transpose(0, 2, 1)
    ck = jnp.pad(ct, ((0, 0), (0, DP - D), (0, DP - D)))
    cb = jnp.pad(en2_b.reshape(D, D).T, ((0, DP - D), (0, DP - D)))[None]
    w2t = jnp.concatenate([ck, cb], axis=0).reshape(KP, DP)

    rootp = pad_w(root)
    cbp = pad_b(conv_b)
    mw1 = pad_w(msg_W[:D])
    mw2 = pad_w(msg_W[D:])
    mbp = pad_b(msg_b)

    src2d = edge_index[0].reshape(N_EDGES // GATHER_CHUNK, GATHER_CHUNK)
    dst3 = edge_index[1].reshape(SCAT_NROWS, SCAT_ROW)
    zeros_nd = jnp.zeros((N_NODES, DP), f32)

    out = _proj(x, w0, b0, BN)
    ht = _proj_t(edge_attr, w1, b1, BE)

    for step in range(STEPS):
        xj = _sc_gather(out, src2d)
        msgs = _msgs(ht, xj, w2t)
        parts = _sc_scatter(msgs, dst3, zeros_nd)
        if step < STEPS - 1:
            out = _update(parts, out, rootp, cbp, mw1, mw2, mbp)
        else:
            out = _update_final(parts, out, rootp, cbp, mw1, mw2, mbp, x)
    return out


# speedup vs baseline: 1.2055x; 1.0243x over previous
import functools

import jax
import jax.numpy as jnp
from jax import lax
from jax.experimental import pallas as pl
from jax.experimental.pallas import tpu as pltpu
from jax.experimental.pallas import tpu_sc as plsc

N_NODES = 10000
N_EDGES = 160000
D = 42
DP = 48
DB = 64
KP = 43 * DP
STEPS = 3

SC_CORES = 2
SC_SUBCORES = 16
SC_WORKERS = SC_CORES * SC_SUBCORES

SCAT_ROW = 125
SCAT_NROWS = N_EDGES // SCAT_ROW
SCAT_WROWS = SCAT_NROWS // SC_WORKERS
SCAT_GRP = 10
SCAT_NGRP = SCAT_WROWS // SCAT_GRP

GATHER_CHUNK = 128
GATHER_MULT = 5
GATHER_WIN = GATHER_CHUNK * GATHER_MULT

BE = 6400
BN = 1000

_F32 = jnp.float32


def _proj_body(x_ref, w_ref, b_ref, o_ref):
    o_ref[...] = jnp.maximum(
        jnp.dot(x_ref[...], w_ref[...], preferred_element_type=_F32)
        + b_ref[...], 0.0)


def _proj(x, w, b, blk):
    rows = x.shape[0]
    return pl.pallas_call(
        _proj_body,
        grid=(rows // blk,),
        in_specs=[
            pl.BlockSpec((blk, x.shape[1]), lambda i: (i, 0)),
            pl.BlockSpec(w.shape, lambda i: (0, 0)),
            pl.BlockSpec(b.shape, lambda i: (0, 0)),
        ],
        out_specs=pl.BlockSpec((blk, DP), lambda i: (i, 0)),
        out_shape=jax.ShapeDtypeStruct((rows, DP), _F32),
    )(x, w, b)


def _proj_t_body(x_ref, w_ref, b_ref, o_ref):
    o_ref[...] = jnp.maximum(
        jnp.dot(x_ref[...], w_ref[...], preferred_element_type=_F32)
        + b_ref[...], 0.0).T


def _proj_t(x, w, b, blk):
    rows = x.shape[0]
    return pl.pallas_call(
        _proj_t_body,
        grid=(rows // blk,),
        in_specs=[
            pl.BlockSpec((blk, x.shape[1]), lambda i: (i, 0)),
            pl.BlockSpec(w.shape, lambda i: (0, 0)),
            pl.BlockSpec(b.shape, lambda i: (0, 0)),
        ],
        out_specs=pl.BlockSpec((DP, blk), lambda i: (0, i)),
        out_shape=jax.ShapeDtypeStruct((DP, rows), _F32),
    )(x, w, b)


def _msg_body(ht_ref, xj_ref, w2t_ref, o_ref):
    ut = lax.dot_general(w2t_ref[...], xj_ref[...],
                         (((1,), (1,)), ((), ())),
                         preferred_element_type=_F32)
    u3 = ut.reshape(43, DP, BE)
    ht = ht_ref[...]
    acc = u3[0] * ht[0][None, :]
    for k in range(1, 43):
        acc = acc + u3[k] * ht[k][None, :]
    o_ref[...] = acc.T


def _msgs(ht, xj, w2t):
    return pl.pallas_call(
        _msg_body,
        grid=(N_EDGES // BE,),
        in_specs=[
            pl.BlockSpec((DP, BE), lambda i: (0, i)),
            pl.BlockSpec((BE, DP), lambda i: (i, 0)),
            pl.BlockSpec((KP, DP), lambda i: (0, 0)),
        ],
        out_specs=pl.BlockSpec((BE, DP), lambda i: (i, 0)),
        out_shape=jax.ShapeDtypeStruct((N_EDGES, DP), _F32),
    )(ht, xj, w2t)


def _upd_core(ap_ref, prev_ref, root_ref, cb_ref, w1_ref, w2_ref, mb_ref):
    aggr = ap_ref[0] + ap_ref[1]
    prev = prev_ref[...]
    conv = aggr + jnp.dot(prev, root_ref[...], preferred_element_type=_F32) + cb_ref[...]
    m = jnp.maximum(conv, 0.0)
    return (jnp.dot(m, w1_ref[...], preferred_element_type=_F32)
            + jnp.dot(prev, w2_ref[...], preferred_element_type=_F32)
            + mb_ref[...])


def _upd_body(ap_ref, prev_ref, root_ref, cb_ref, w1_ref, w2_ref, mb_ref, o_ref):
    o_ref[...] = _upd_core(ap_ref, prev_ref, root_ref, cb_ref, w1_ref, w2_ref, mb_ref)


def _upd_final_body(ap_ref, prev_ref, root_ref, cb_ref, w1_ref, w2_ref, mb_ref,
                    init_ref, o_ref):
    full = _upd_core(ap_ref, prev_ref, root_ref, cb_ref, w1_ref, w2_ref, mb_ref)
    o_ref[...] = full[:, :D] + init_ref[...]


def _upd_weight_specs():
    return [
        pl.BlockSpec((2, BN, DP), lambda i: (0, i, 0)),
        pl.BlockSpec((BN, DP), lambda i: (i, 0)),
        pl.BlockSpec((DP, DP), lambda i: (0, 0)),
        pl.BlockSpec((1, DP), lambda i: (0, 0)),
        pl.BlockSpec((DP, DP), lambda i: (0, 0)),
        pl.BlockSpec((DP, DP), lambda i: (0, 0)),
        pl.BlockSpec((1, DP), lambda i: (0, 0)),
    ]


def _update(parts, prev, rootp, cbp, mw1, mw2, mbp):
    return pl.pallas_call(
        _upd_body,
        grid=(N_NODES // BN,),
        in_specs=_upd_weight_specs(),
        out_specs=pl.BlockSpec((BN, DP), lambda i: (i, 0)),
        out_shape=jax.ShapeDtypeStruct((N_NODES, DP), _F32),
    )(parts, prev, rootp, cbp, mw1, mw2, mbp)


def _update_final(parts, prev, rootp, cbp, mw1, mw2, mbp, init):
    return pl.pallas_call(
        _upd_final_body,
        grid=(N_NODES // BN,),
        in_specs=_upd_weight_specs() + [pl.BlockSpec((BN, D), lambda i: (i, 0))],
        out_specs=pl.BlockSpec((BN, D), lambda i: (i, 0)),
        out_shape=jax.ShapeDtypeStruct((N_NODES, D), _F32),
    )(parts, prev, rootp, cbp, mw1, mw2, mbp, init)


def _sc_mesh():
    return plsc.VectorSubcoreMesh(core_axis_name="core", subcore_axis_name="subcore")


_SC_PARAMS = pltpu.CompilerParams(use_tc_tiling_on_sc=False)


def _sc_gather(table, idx2d):
    @functools.partial(
        pl.kernel,
        out_type=jax.ShapeDtypeStruct((N_EDGES, DP), _F32),
        mesh=_sc_mesh(),
        compiler_params=_SC_PARAMS,
    )
    def k(tab_hbm, i_hbm, o_hbm):
        def body(i_vmem, o_vmem):
            for m in range(GATHER_MULT):
                pltpu.sync_copy(tab_hbm.at[i_vmem.at[m]],
                                o_vmem.at[pl.ds(m * GATHER_CHUNK, GATHER_CHUNK)])

        pltpu.emit_pipeline(
            body,
            grid=(N_EDGES // GATHER_WIN,),
            in_specs=[pl.BlockSpec((GATHER_MULT, GATHER_CHUNK), lambda i: (i, 0))],
            out_specs=[pl.BlockSpec((GATHER_WIN, DP), lambda i: (i, 0))],
            core_axis_name=("core", "subcore"),
            dimension_semantics=(pltpu.PARALLEL,),
        )(i_hbm, o_hbm)

    return k(table, idx2d)


def _sc_scatter(msgs, dst3, zeros_nd):
    @functools.partial(
        pl.kernel,
        out_type=jax.ShapeDtypeStruct((SC_CORES, N_NODES, DP), _F32),
        mesh=_sc_mesh(),
        scratch_types=[
            pltpu.VMEM((SCAT_GRP, SCAT_ROW), jnp.int32),
            pltpu.VMEM((SCAT_GRP * SCAT_ROW, DP), _F32),
            pltpu.VMEM_SHARED((N_NODES, DP), _F32),
        ],
        compiler_params=_SC_PARAMS,
    )
    def k(msgs_hbm, dst_hbm, z_hbm, o_hbm, idx_v, rows_v, aggr_sh):
        c = lax.axis_index("core")
        s = lax.axis_index("subcore")
        stripe = N_NODES // SC_SUBCORES

        pltpu.sync_copy(z_hbm.at[pl.ds(s * stripe, stripe)],
                        aggr_sh.at[pl.ds(s * stripe, stripe)])
        plsc.subcore_barrier()

        w = c * SC_SUBCORES + s
        row0 = w * SCAT_WROWS

        @pl.loop(0, SCAT_NGRP)
        def _(g):
            base = row0 + g * SCAT_GRP
            pltpu.sync_copy(dst_hbm.at[pl.ds(base, SCAT_GRP)], idx_v)
            pltpu.sync_copy(msgs_hbm.at[pl.ds(base * SCAT_ROW, SCAT_GRP * SCAT_ROW)],
                            rows_v)
            for m in range(SCAT_GRP):
                pltpu.sync_copy(rows_v.at[pl.ds(m * SCAT_ROW, SCAT_ROW)],
                                aggr_sh.at[idx_v.at[m]], add=True)

        plsc.subcore_barrier()
        pltpu.sync_copy(aggr_sh.at[pl.ds(s * stripe, stripe)],
                        o_hbm.at[c, pl.ds(s * stripe, stripe)])

    return k(msgs, dst3, zeros_nd)


def kernel(x, edge_index, edge_attr, lin0_W, lin0_b, en1_W, en1_b, en2_W, en2_b,
           root, conv_b, msg_W, msg_b):
    f32 = _F32
    pad_w = lambda w: jnp.pad(w.astype(f32), ((0, DP - w.shape[0]), (0, DP - w.shape[1])))
    pad_b = lambda b: jnp.pad(b.astype(f32), (0, DP - b.shape[0]))[None, :]

    w0 = jnp.pad(lin0_W, ((0, 0), (0, DP - D)))
    b0 = pad_b(lin0_b)
    w1 = jnp.pad(en1_W, ((0, 0), (0, DP - D)))
    b1 = pad_b(en1_b).at[0, D].set(1.0)

    ct = en2_W.reshape(D, D, D).transpose(0, 2, 1)
    ck = jnp.pad(ct, ((0, 0), (0, DP - D), (0, DP - D)))
    cb = jnp.pad(en2_b.reshape(D, D).T, ((0, DP - D), (0, DP - D)))[None]
    w2t = jnp.concatenate([ck, cb], axis=0).reshape(KP, DP)

    rootp = pad_w(root)
    cbp = pad_b(conv_b)
    mw1 = pad_w(msg_W[:D])
    mw2 = pad_w(msg_W[D:])
    mbp = pad_b(msg_b)

    src2d = edge_index[0].reshape(N_EDGES // GATHER_CHUNK, GATHER_CHUNK)
    dst3 = edge_index[1].reshape(SCAT_NROWS, SCAT_ROW)
    zeros_nd = jnp.zeros((N_NODES, DP), f32)

    out = _proj(x, w0, b0, BN)
    ht = _proj_t(edge_attr, w1, b1, BE)

    for step in range(STEPS):
        xj = _sc_gather(out, src2d)
        msgs = _msgs(ht, xj, w2t)
        parts = _sc_scatter(msgs, dst3, zeros_nd)
        if step < STEPS - 1:
            out = _update(parts, out, rootp, cbp, mw1, mw2, mbp)
        else:
            out = _update_final(parts, out, rootp, cbp, mw1, mw2, mbp, x)
    return out


# speedup vs baseline: 1.2058x; 1.0002x over previous
import functools

import jax
import jax.numpy as jnp
from jax import lax
from jax.experimental import pallas as pl
from jax.experimental.pallas import tpu as pltpu
from jax.experimental.pallas import tpu_sc as plsc

N_NODES = 10000
N_EDGES = 160000
D = 42
DP = 48
DB = 64
KP = 43 * DP
STEPS = 3

SC_CORES = 2
SC_SUBCORES = 16
SC_WORKERS = SC_CORES * SC_SUBCORES

SCAT_ROW = 125
SCAT_NROWS = N_EDGES // SCAT_ROW
SCAT_WROWS = SCAT_NROWS // SC_WORKERS
SCAT_GRP = 10
SCAT_NGRP = SCAT_WROWS // SCAT_GRP

GATHER_CHUNK = 128
GATHER_MULT = 10
GATHER_WIN = GATHER_CHUNK * GATHER_MULT

BE = 6400
BN = 1000

_F32 = jnp.float32


def _proj_body(x_ref, w_ref, b_ref, o_ref):
    o_ref[...] = jnp.maximum(
        jnp.dot(x_ref[...], w_ref[...], preferred_element_type=_F32)
        + b_ref[...], 0.0)


def _proj(x, w, b, blk):
    rows = x.shape[0]
    return pl.pallas_call(
        _proj_body,
        grid=(rows // blk,),
        in_specs=[
            pl.BlockSpec((blk, x.shape[1]), lambda i: (i, 0)),
            pl.BlockSpec(w.shape, lambda i: (0, 0)),
            pl.BlockSpec(b.shape, lambda i: (0, 0)),
        ],
        out_specs=pl.BlockSpec((blk, DP), lambda i: (i, 0)),
        out_shape=jax.ShapeDtypeStruct((rows, DP), _F32),
    )(x, w, b)


def _proj_t_body(x_ref, w_ref, b_ref, o_ref):
    o_ref[...] = jnp.maximum(
        jnp.dot(x_ref[...], w_ref[...], preferred_element_type=_F32)
        + b_ref[...], 0.0).T


def _proj_t(x, w, b, blk):
    rows = x.shape[0]
    return pl.pallas_call(
        _proj_t_body,
        grid=(rows // blk,),
        in_specs=[
            pl.BlockSpec((blk, x.shape[1]), lambda i: (i, 0)),
            pl.BlockSpec(w.shape, lambda i: (0, 0)),
            pl.BlockSpec(b.shape, lambda i: (0, 0)),
        ],
        out_specs=pl.BlockSpec((DP, blk), lambda i: (0, i)),
        out_shape=jax.ShapeDtypeStruct((DP, rows), _F32),
    )(x, w, b)


def _msg_body(ht_ref, xj_ref, w2t_ref, o_ref):
    ut = lax.dot_general(w2t_ref[...], xj_ref[...],
                         (((1,), (1,)), ((), ())),
                         preferred_element_type=_F32)
    u3 = ut.reshape(43, DP, BE)
    ht = ht_ref[...]
    acc = u3[0] * ht[0][None, :]
    for k in range(1, 43):
        acc = acc + u3[k] * ht[k][None, :]
    o_ref[...] = acc.T


def _msgs(ht, xj, w2t):
    return pl.pallas_call(
        _msg_body,
        grid=(N_EDGES // BE,),
        in_specs=[
            pl.BlockSpec((DP, BE), lambda i: (0, i)),
            pl.BlockSpec((BE, DP), lambda i: (i, 0)),
            pl.BlockSpec((KP, DP), lambda i: (0, 0)),
        ],
        out_specs=pl.BlockSpec((BE, DP), lambda i: (i, 0)),
        out_shape=jax.ShapeDtypeStruct((N_EDGES, DP), _F32),
    )(ht, xj, w2t)


def _upd_core(ap_ref, prev_ref, root_ref, cb_ref, w1_ref, w2_ref, mb_ref):
    aggr = ap_ref[0] + ap_ref[1]
    prev = prev_ref[...]
    conv = aggr + jnp.dot(prev, root_ref[...], preferred_element_type=_F32) + cb_ref[...]
    m = jnp.maximum(conv, 0.0)
    return (jnp.dot(m, w1_ref[...], preferred_element_type=_F32)
            + jnp.dot(prev, w2_ref[...], preferred_element_type=_F32)
            + mb_ref[...])


def _upd_body(ap_ref, prev_ref, root_ref, cb_ref, w1_ref, w2_ref, mb_ref, o_ref):
    o_ref[...] = _upd_core(ap_ref, prev_ref, root_ref, cb_ref, w1_ref, w2_ref, mb_ref)


def _upd_final_body(ap_ref, prev_ref, root_ref, cb_ref, w1_ref, w2_ref, mb_ref,
                    init_ref, o_ref):
    full = _upd_core(ap_ref, prev_ref, root_ref, cb_ref, w1_ref, w2_ref, mb_ref)
    o_ref[...] = full[:, :D] + init_ref[...]


def _upd_weight_specs():
    return [
        pl.BlockSpec((2, BN, DP), lambda i: (0, i, 0)),
        pl.BlockSpec((BN, DP), lambda i: (i, 0)),
        pl.BlockSpec((DP, DP), lambda i: (0, 0)),
        pl.BlockSpec((1, DP), lambda i: (0, 0)),
        pl.BlockSpec((DP, DP), lambda i: (0, 0)),
        pl.BlockSpec((DP, DP), lambda i: (0, 0)),
        pl.BlockSpec((1, DP), lambda i: (0, 0)),
    ]


def _update(parts, prev, rootp, cbp, mw1, mw2, mbp):
    return pl.pallas_call(
        _upd_body,
        grid=(N_NODES // BN,),
        in_specs=_upd_weight_specs(),
        out_specs=pl.BlockSpec((BN, DP), lambda i: (i, 0)),
        out_shape=jax.ShapeDtypeStruct((N_NODES, DP), _F32),
    )(parts, prev, rootp, cbp, mw1, mw2, mbp)


def _update_final(parts, prev, rootp, cbp, mw1, mw2, mbp, init):
    return pl.pallas_call(
        _upd_final_body,
        grid=(N_NODES // BN,),
        in_specs=_upd_weight_specs() + [pl.BlockSpec((BN, D), lambda i: (i, 0))],
        out_specs=pl.BlockSpec((BN, D), lambda i: (i, 0)),
        out_shape=jax.ShapeDtypeStruct((N_NODES, D), _F32),
    )(parts, prev, rootp, cbp, mw1, mw2, mbp, init)


def _sc_mesh():
    return plsc.VectorSubcoreMesh(core_axis_name="core", subcore_axis_name="subcore")


_SC_PARAMS = pltpu.CompilerParams(use_tc_tiling_on_sc=False)


def _sc_gather(table, idx2d):
    @functools.partial(
        pl.kernel,
        out_type=jax.ShapeDtypeStruct((N_EDGES, DP), _F32),
        mesh=_sc_mesh(),
        compiler_params=_SC_PARAMS,
    )
    def k(tab_hbm, i_hbm, o_hbm):
        def body(i_vmem, o_vmem):
            for m in range(GATHER_MULT):
                pltpu.sync_copy(tab_hbm.at[i_vmem.at[m]],
                                o_vmem.at[pl.ds(m * GATHER_CHUNK, GATHER_CHUNK)])

        pltpu.emit_pipeline(
            body,
            grid=(N_EDGES // GATHER_WIN,),
            in_specs=[pl.BlockSpec((GATHER_MULT, GATHER_CHUNK), lambda i: (i, 0))],
            out_specs=[pl.BlockSpec((GATHER_WIN, DP), lambda i: (i, 0))],
            core_axis_name=("core", "subcore"),
            dimension_semantics=(pltpu.PARALLEL,),
        )(i_hbm, o_hbm)

    return k(table, idx2d)


def _sc_scatter(msgs, dst3, zeros_nd):
    @functools.partial(
        pl.kernel,
        out_type=jax.ShapeDtypeStruct((SC_CORES, N_NODES, DP), _F32),
        mesh=_sc_mesh(),
        scratch_types=[
            pltpu.VMEM((SCAT_GRP, SCAT_ROW), jnp.int32),
            pltpu.VMEM((SCAT_GRP * SCAT_ROW, DP), _F32),
            pltpu.VMEM_SHARED((N_NODES, DP), _F32),
        ],
        compiler_params=_SC_PARAMS,
    )
    def k(msgs_hbm, dst_hbm, z_hbm, o_hbm, idx_v, rows_v, aggr_sh):
        c = lax.axis_index("core")
        s = lax.axis_index("subcore")
        stripe = N_NODES // SC_SUBCORES

        pltpu.sync_copy(z_hbm.at[pl.ds(s * stripe, stripe)],
                        aggr_sh.at[pl.ds(s * stripe, stripe)])
        plsc.subcore_barrier()

        w = c * SC_SUBCORES + s
        row0 = w * SCAT_WROWS

        @pl.loop(0, SCAT_NGRP)
        def _(g):
            base = row0 + g * SCAT_GRP
            pltpu.sync_copy(dst_hbm.at[pl.ds(base, SCAT_GRP)], idx_v)
            pltpu.sync_copy(msgs_hbm.at[pl.ds(base * SCAT_ROW, SCAT_GRP * SCAT_ROW)],
                            rows_v)
            for m in range(SCAT_GRP):
                pltpu.sync_copy(rows_v.at[pl.ds(m * SCAT_ROW, SCAT_ROW)],
                                aggr_sh.at[idx_v.at[m]], add=True)

        plsc.subcore_barrier()
        pltpu.sync_copy(aggr_sh.at[pl.ds(s * stripe, stripe)],
                        o_hbm.at[c, pl.ds(s * stripe, stripe)])

    return k(msgs, dst3, zeros_nd)


def kernel(x, edge_index, edge_attr, lin0_W, lin0_b, en1_W, en1_b, en2_W, en2_b,
           root, conv_b, msg_W, msg_b):
    f32 = _F32
    pad_w = lambda w: jnp.pad(w.astype(f32), ((0, DP - w.shape[0]), (0, DP - w.shape[1])))
    pad_b = lambda b: jnp.pad(b.astype(f32), (0, DP - b.shape[0]))[None, :]

    w0 = jnp.pad(lin0_W, ((0, 0), (0, DP - D)))
    b0 = pad_b(lin0_b)
    w1 = jnp.pad(en1_W, ((0, 0), (0, DP - D)))
    b1 = pad_b(en1_b).at[0, D].set(1.0)

    ct = en2_W.reshape(D, D, D).transpose(0, 2, 1)
    ck = jnp.pad(ct, ((0, 0), (0, DP - D), (0, DP - D)))
    cb = jnp.pad(en2_b.reshape(D, D).T, ((0, DP - D), (0, DP - D)))[None]
    w2t = jnp.concatenate([ck, cb], axis=0).reshape(KP, DP)

    rootp = pad_w(root)
    cbp = pad_b(conv_b)
    mw1 = pad_w(msg_W[:D])
    mw2 = pad_w(msg_W[D:])
    mbp = pad_b(msg_b)

    src2d = edge_index[0].reshape(N_EDGES // GATHER_CHUNK, GATHER_CHUNK)
    dst3 = edge_index[1].reshape(SCAT_NROWS, SCAT_ROW)
    zeros_nd = jnp.zeros((N_NODES, DP), f32)

    out = _proj(x, w0, b0, BN)
    ht = _proj_t(edge_attr, w1, b1, BE)

    for step in range(STEPS):
        xj = _sc_gather(out, src2d)
        msgs = _msgs(ht, xj, w2t)
        parts = _sc_scatter(msgs, dst3, zeros_nd)
        if step < STEPS - 1:
            out = _update(parts, out, rootp, cbp, mw1, mw2, mbp)
        else:
            out = _update_final(parts, out, rootp, cbp, mw1, mw2, mbp, x)
    return out
